# baseline (speedup 1.0000x reference)
import functools
import jax
import jax.numpy as jnp
from jax import lax
from jax.experimental import pallas as pl
from jax.experimental.pallas import tpu as pltpu, tpu_sc as plsc

B = 4096
N = 50
L = N - 1
LAMBDA = 0.5

RB = 64
ROWS = B // 128 * 4
GRID = ROWS // RB

_NC, _NS, _NL = 2, 16, 16
NW = _NC * _NS
TOT_L = B * L * 4
PW = TOT_L // NW
CH = 3136
NCH = PW // CH

_sc_mesh = plsc.VectorSubcoreMesh(core_axis_name="c", subcore_axis_name="s")


@functools.partial(
    pl.kernel, mesh=_sc_mesh,
    out_type=jax.ShapeDtypeStruct((NW, 16), jnp.float32),
    scratch_types=[
        pltpu.VMEM((CH,), jnp.float32),
        pltpu.VMEM((CH,), jnp.float32),
        pltpu.VMEM((16,), jnp.float32),
    ],
)
def _line_mse_sc(a_hbm, b_hbm, out_hbm, av, bv, accv):
    wid = lax.axis_index("s") * _NC + lax.axis_index("c")
    base = wid * PW

    def body(i, acc):
        pltpu.sync_copy(a_hbm.at[pl.ds(base + i * CH, CH)], av)
        pltpu.sync_copy(b_hbm.at[pl.ds(base + i * CH, CH)], bv)

        def inner(j, acc2):
            d = av[pl.ds(j * 16, 16)] - bv[pl.ds(j * 16, 16)]
            return acc2 + d * d

        return lax.fori_loop(0, CH // 16, inner, acc)

    acc = lax.fori_loop(0, NCH, body, jnp.zeros((16,), jnp.float32))
    accv[...] = acc
    pltpu.sync_copy(accv, out_hbm.at[wid])


def _loss_kernel(pn_ref, gn_ref, pl_ref, lp_ref, out_ref):
    x = pn_ref[...]
    g = gn_ref[...]
    y = pl_ref[...]
    z = lp_ref[...]

    dn = x - g
    s1 = jnp.sum(dn * dn)

    ys = jnp.concatenate([y[:, 2:, :], jnp.zeros((L, 2, 128), jnp.float32)],
                         axis=1)
    ysn = jnp.concatenate([ys[1:L], jnp.zeros((1, RB, 128), jnp.float32)],
                          axis=0)
    err = ys - ysn - x[1:N]
    riota = jax.lax.broadcasted_iota(jnp.int32, (1, RB, 1), 1)
    s3 = jnp.sum(jnp.where(riota % 4 < 2, err * err, 0.0))

    u = z * ys
    gsum = u + jnp.concatenate([u[:, 1:, :],
                                jnp.zeros((L, 1, 128), jnp.float32)], axis=1)
    xs2 = jnp.concatenate([x[:, 2:, :], jnp.zeros((N, 2, 128), jnp.float32)],
                          axis=1)
    v2 = xs2 * xs2
    dv2 = v2[0:L] - v2[1:N]
    lf = 2.0 * gsum - dv2
    s4 = jnp.sum(jnp.where(riota % 4 == 0, lf * lf, 0.0))

    lane = jax.lax.broadcasted_iota(jnp.int32, (1, 1, 128), 2)
    packed = (jnp.where(lane == 0, s1, 0.0)
              + jnp.where(lane == 2, s3, 0.0) + jnp.where(lane == 3, s4, 0.0))

    @pl.when(pl.program_id(0) == 0)
    def _init():
        out_ref[...] = packed

    @pl.when(pl.program_id(0) != 0)
    def _acc():
        out_ref[...] = out_ref[...] + packed


def _to_t(x, n):
    return (x.transpose(1, 2, 0).reshape(n, 4, B // 128, 128)
            .transpose(0, 2, 1, 3).reshape(n, B // 128 * 4, 128))


def kernel(pred_node, gt_node, pred_line, gt_line, adj, line_param, node_count):
    del adj, node_count
    pn = _to_t(pred_node, N)
    gn = _to_t(gt_node, N)
    pline = _to_t(pred_line, L)
    lpar = _to_t(line_param, L)

    pl_flat = pline.reshape(TOT_L)
    gl_flat = _to_t(gt_line, L).reshape(TOT_L)
    sc_out = _line_mse_sc(pl_flat, gl_flat)

    spec_n = pl.BlockSpec((N, RB, 128), lambda i: (0, i, 0))
    spec_l = pl.BlockSpec((L, RB, 128), lambda i: (0, i, 0))

    sums = pl.pallas_call(
        _loss_kernel,
        grid=(GRID,),
        in_specs=[spec_n, spec_n, spec_l, spec_l],
        out_specs=pl.BlockSpec((1, 1, 128), lambda i: (0, 0, 0)),
        out_shape=jax.ShapeDtypeStruct((1, 1, 128), jnp.float32),
    )(pn, gn, pline, lpar)

    s1 = sums[0, 0, 0]
    s2 = jnp.sum(sc_out)
    s3 = sums[0, 0, 2]
    s4 = sums[0, 0, 3]

    node_mse = s1 / (B * N * 4)
    line_mse = s2 / (B * L * 4)
    pred_loss = node_mse + line_mse
    physics_loss = s3 / (B * N * 2) + s4 / (B * L)
    total_loss = pred_loss + LAMBDA * physics_loss
    return (total_loss, pred_loss, physics_loss)
